# baseline (speedup 1.0000x reference)
import functools

import jax
import jax.numpy as jnp
from jax import lax
from jax.experimental import pallas as pl
from jax.experimental.pallas import tpu as pltpu
from jax.experimental.pallas import tpu_sc as plsc

_COMMIT_W = 0.25


def _vq_body(x_ref, cb_ref, ind_ref, loss_ref, perp_ref,
             cluster_acc, loss_acc, *, n_total, k, d):
    i = pl.program_id(0)
    nsteps = pl.num_programs(0)

    x = x_ref[...]
    cb = cb_ref[...]

    x_sq = jnp.sum(x * x, axis=-1, keepdims=True)
    c_sq = jnp.sum(cb * cb, axis=-1)[None, :]
    scores = jax.lax.dot_general(x, cb, (((1,), (1,)), ((), ())),
                                 preferred_element_type=jnp.float32)
    d2 = x_sq + c_sq - 2.0 * scores
    dist = jnp.sqrt(jnp.clip(d2, 0.0, None))
    m = jnp.min(dist, axis=1, keepdims=True)
    iota_k = jax.lax.broadcasted_iota(jnp.int32, dist.shape, 1)
    ind = jnp.min(jnp.where(dist == m, iota_k, jnp.int32(k)), axis=1)

    oh = (iota_k == ind[:, None]).astype(jnp.float32)
    ind_ref[...] = ind.reshape(ind_ref.shape)

    @pl.when(i == 0)
    def _init():
        cluster_acc[...] = jnp.zeros_like(cluster_acc)
        loss_acc[0, 0] = 0.0

    loss_acc[0, 0] += jnp.sum(m * m)
    cluster_acc[...] += jnp.sum(oh, axis=0, keepdims=True)

    @pl.when(i == nsteps - 1)
    def _fini():
        cs = cluster_acc[...]
        probs = cs / jnp.sum(cs)
        ent = -jnp.sum(probs * jnp.log(probs + 1e-10))
        perp_ref[0, 0] = jnp.exp(ent)
        loss_ref[0, 0] = loss_acc[0, 0] * ((1.0 + _COMMIT_W) / (n_total * d))


def _tc_stage(x, codebook, bn):
    n, d = x.shape
    k = codebook.shape[0]
    nb = n // bn
    body = functools.partial(_vq_body, n_total=n, k=k, d=d)
    return pl.pallas_call(
        body,
        grid=(nb,),
        in_specs=[
            pl.BlockSpec((bn, d), lambda i: (i, 0)),
            pl.BlockSpec((k, d), lambda i: (0, 0)),
        ],
        out_specs=[
            pl.BlockSpec((bn,), lambda i: (i,)),
            pl.BlockSpec(memory_space=pltpu.SMEM),
            pl.BlockSpec(memory_space=pltpu.SMEM),
        ],
        out_shape=[
            jax.ShapeDtypeStruct((n,), jnp.int32),
            jax.ShapeDtypeStruct((1, 1), jnp.float32),
            jax.ShapeDtypeStruct((1, 1), jnp.float32),
        ],
        scratch_shapes=[
            pltpu.VMEM((1, k), jnp.float32),
            pltpu.SMEM((1, 1), jnp.float32),
        ],
        compiler_params=pltpu.CompilerParams(
            dimension_semantics=("arbitrary",)),
    )(x, codebook)


def _sc_gather(codebook, ind_flat, n, d):
    info = plsc.get_sparse_core_info()
    nw = info.num_cores * info.num_subcores
    rpw = n // nw
    chunks = []
    off = 0
    while off < rpw:
        sz = min(128, rpw - off)
        chunks.append((off, sz))
        off += sz

    @functools.partial(
        pl.kernel,
        mesh=plsc.VectorSubcoreMesh(core_axis_name="c", subcore_axis_name="s"),
        out_type=jax.ShapeDtypeStruct((n, d), jnp.float32),
        scratch_types=[
            pltpu.VMEM((rpw,), jnp.int32),
            pltpu.VMEM((rpw, d), jnp.float32),
            pltpu.SemaphoreType.DMA,
        ],
        compiler_params=pltpu.CompilerParams(use_tc_tiling_on_sc=False),
    )
    def gather_kernel(cb_hbm, ind_hbm, out_hbm, idx_v, rows_v, sem):
        wid = lax.axis_index("s") * info.num_cores + lax.axis_index("c")
        base = wid * rpw
        pltpu.sync_copy(ind_hbm.at[pl.ds(base, rpw)], idx_v)
        copies = []
        for off, sz in chunks:
            copies.append(pltpu.async_copy(
                cb_hbm.at[idx_v.at[pl.ds(off, sz)]],
                rows_v.at[pl.ds(off, sz)], sem))
        for c in copies:
            c.wait()
        pltpu.sync_copy(rows_v, out_hbm.at[pl.ds(base, rpw)])

    return gather_kernel(codebook, ind_flat)


def kernel(z, codebook):
    b, t, d = z.shape
    x = z.reshape(-1, d)
    n = x.shape[0]

    ind_flat, loss, perp = _tc_stage(x, codebook, bn=4096)
    out = _sc_gather(codebook, ind_flat, n, d)

    return (out.reshape(b, t, d), ind_flat.reshape(b, t),
            loss[0, 0], perp[0, 0])


# speedup vs baseline: 1.0080x; 1.0080x over previous
import functools

import jax
import jax.numpy as jnp
from jax import lax
from jax.experimental import pallas as pl
from jax.experimental.pallas import tpu as pltpu
from jax.experimental.pallas import tpu_sc as plsc

_COMMIT_W = 0.25


def _vq_body(x_ref, cb_ref, ind_ref, loss_ref, perp_ref,
             cluster_acc, loss_acc, *, n_total, k, d):
    i = pl.program_id(0)
    nsteps = pl.num_programs(0)

    x = x_ref[...]
    cb = cb_ref[...]

    x_sq = jnp.sum(x * x, axis=-1, keepdims=True)
    c_sq = jnp.sum(cb * cb, axis=-1)[None, :]
    scores = jax.lax.dot_general(x, cb, (((1,), (1,)), ((), ())),
                                 preferred_element_type=jnp.float32)
    d2 = x_sq + c_sq - 2.0 * scores
    dist = jnp.sqrt(jnp.clip(d2, 0.0, None))
    m = jnp.min(dist, axis=1, keepdims=True)
    iota_k = jax.lax.broadcasted_iota(jnp.int32, dist.shape, 1)
    ind = jnp.min(jnp.where(dist == m, iota_k, jnp.int32(k)), axis=1)

    oh = (iota_k == ind[:, None]).astype(jnp.float32)
    ind_ref[...] = ind.reshape(ind_ref.shape)

    @pl.when(i == 0)
    def _init():
        cluster_acc[...] = jnp.zeros_like(cluster_acc)
        loss_acc[0, 0] = 0.0

    loss_acc[0, 0] += jnp.sum(m * m)
    cluster_acc[...] += jnp.sum(oh, axis=0, keepdims=True)

    @pl.when(i == nsteps - 1)
    def _fini():
        cs = cluster_acc[...]
        probs = cs / jnp.sum(cs)
        ent = -jnp.sum(probs * jnp.log(probs + 1e-10))
        perp_ref[0, 0] = jnp.exp(ent)
        loss_ref[0, 0] = loss_acc[0, 0] * ((1.0 + _COMMIT_W) / (n_total * d))


def _tc_stage(x, codebook, bn):
    n, d = x.shape
    k = codebook.shape[0]
    nb = n // bn
    body = functools.partial(_vq_body, n_total=n, k=k, d=d)
    return pl.pallas_call(
        body,
        grid=(nb,),
        in_specs=[
            pl.BlockSpec((bn, d), lambda i: (i, 0)),
            pl.BlockSpec((k, d), lambda i: (0, 0)),
        ],
        out_specs=[
            pl.BlockSpec((1, 1, bn), lambda i: (i, 0, 0)),
            pl.BlockSpec(memory_space=pltpu.SMEM),
            pl.BlockSpec(memory_space=pltpu.SMEM),
        ],
        out_shape=[
            jax.ShapeDtypeStruct((nb, 1, bn), jnp.int32),
            jax.ShapeDtypeStruct((1, 1), jnp.float32),
            jax.ShapeDtypeStruct((1, 1), jnp.float32),
        ],
        scratch_shapes=[
            pltpu.VMEM((1, k), jnp.float32),
            pltpu.SMEM((1, 1), jnp.float32),
        ],
        compiler_params=pltpu.CompilerParams(
            dimension_semantics=("arbitrary",)),
    )(x, codebook)


def _sc_gather(codebook, ind_flat, n, d):
    info = plsc.get_sparse_core_info()
    nw = info.num_cores * info.num_subcores
    rpw = n // nw
    chunks = []
    off = 0
    while off < rpw:
        sz = min(128, rpw - off)
        chunks.append((off, sz))
        off += sz

    @functools.partial(
        pl.kernel,
        mesh=plsc.VectorSubcoreMesh(core_axis_name="c", subcore_axis_name="s"),
        out_type=jax.ShapeDtypeStruct((n, d), jnp.float32),
        scratch_types=[
            pltpu.VMEM((rpw,), jnp.int32),
            pltpu.VMEM((rpw, d), jnp.float32),
            pltpu.SemaphoreType.DMA,
        ],
        compiler_params=pltpu.CompilerParams(use_tc_tiling_on_sc=False),
    )
    def gather_kernel(cb_hbm, ind_hbm, out_hbm, idx_v, rows_v, sem):
        wid = lax.axis_index("s") * info.num_cores + lax.axis_index("c")
        base = wid * rpw
        pltpu.sync_copy(ind_hbm.at[pl.ds(base, rpw)], idx_v)
        copies = []
        for off, sz in chunks:
            copies.append(pltpu.async_copy(
                cb_hbm.at[idx_v.at[pl.ds(off, sz)]],
                rows_v.at[pl.ds(off, sz)], sem))
        for c in copies:
            c.wait()
        pltpu.sync_copy(rows_v, out_hbm.at[pl.ds(base, rpw)])

    return gather_kernel(codebook, ind_flat)


def kernel(z, codebook):
    b, t, d = z.shape
    x = z.reshape(-1, d)
    n = x.shape[0]

    ind3, loss, perp = _tc_stage(x, codebook, bn=4096)
    ind_flat = ind3.reshape(n)
    out = _sc_gather(codebook, ind_flat, n, d)

    return (out.reshape(b, t, d), ind_flat.reshape(b, t),
            loss[0, 0], perp[0, 0])
